# baseline (speedup 1.0000x reference)
import functools

import jax
import jax.numpy as jnp
from jax import lax
from jax.experimental import pallas as pl
from jax.experimental.pallas import tpu as pltpu
from jax.experimental.pallas import tpu_sc as plsc

NUM_CORES = 2
NUM_SUBCORES = 16
LANES = 16
NW = NUM_CORES * NUM_SUBCORES

BATCH = 16384
EMB = 32
B_W = BATCH // NW
IDX_CHUNK = 128
N_CHUNKS = B_W // IDX_CHUNK
GROUPS = B_W // LANES
PAD = LANES + 1


def _gmf_body(users_h, items_h, ut_h, it_h, w_h, b_h, out_h,
              uidx_v, iidx_v, urows_v, irows_v, tsc_v, outv_v, w_v, b_v, sem):
    wid = lax.axis_index("s") * NUM_CORES + lax.axis_index("c")

    pltpu.sync_copy(users_h.at[wid], uidx_v)
    pltpu.sync_copy(items_h.at[wid], iidx_v)
    pltpu.sync_copy(w_h, w_v)
    pltpu.sync_copy(b_h, b_v)

    copies = []
    for k in range(N_CHUNKS):
        dst = pl.ds(k * IDX_CHUNK, IDX_CHUNK)
        copies.append(pltpu.async_copy(ut_h.at[uidx_v.at[k]], urows_v.at[dst], sem))
        copies.append(pltpu.async_copy(it_h.at[iidx_v.at[k]], irows_v.at[dst], sem))
    for c in copies:
        c.wait()

    w0 = w_v[pl.ds(0, LANES)]
    w1 = w_v[pl.ds(LANES, LANES)]
    b_vec = b_v[...]
    iota = lax.iota(jnp.int32, LANES)

    def group(g, carry):
        base_r = g * LANES
        for j in range(LANES):
            r = base_r + j
            u0 = urows_v[r, pl.ds(0, LANES)]
            u1 = urows_v[r, pl.ds(LANES, LANES)]
            i0 = irows_v[r, pl.ds(0, LANES)]
            i1 = irows_v[r, pl.ds(LANES, LANES)]
            tsc_v[j, pl.ds(0, LANES)] = (u0 * i0) * w0 + (u1 * i1) * w1
        acc = b_vec
        for c in range(LANES):
            col = jnp.full((LANES,), c, jnp.int32)
            acc = acc + plsc.load_gather(tsc_v, [iota, col])
        outv_v[pl.ds(base_r, LANES)] = 1.0 / (1.0 + jnp.exp(-acc))
        return carry

    lax.fori_loop(0, GROUPS, group, 0)
    pltpu.sync_copy(outv_v, out_h.at[wid])


@functools.partial(jax.jit, static_argnames=("interpret",))
def _gmf(users, items, user_table, item_table, w_flat, b_vec, interpret=False):
    run = pl.kernel(
        _gmf_body,
        out_type=jax.ShapeDtypeStruct((NW, B_W), jnp.float32),
        mesh=plsc.VectorSubcoreMesh(core_axis_name="c", subcore_axis_name="s",
                                    num_cores=NUM_CORES, num_subcores=NUM_SUBCORES),
        scratch_types=[
            pltpu.VMEM((N_CHUNKS, IDX_CHUNK), jnp.int32),
            pltpu.VMEM((N_CHUNKS, IDX_CHUNK), jnp.int32),
            pltpu.VMEM((B_W, EMB), jnp.float32),
            pltpu.VMEM((B_W, EMB), jnp.float32),
            pltpu.VMEM((LANES, PAD), jnp.float32),
            pltpu.VMEM((B_W,), jnp.float32),
            pltpu.VMEM((EMB,), jnp.float32),
            pltpu.VMEM((LANES,), jnp.float32),
            pltpu.SemaphoreType.DMA,
        ],
        compiler_params=pltpu.CompilerParams(needs_layout_passes=False,
                                             use_tc_tiling_on_sc=False),
        interpret=interpret,
    )
    return run(users, items, user_table, item_table, w_flat, b_vec)


def kernel(users, items, user_table, item_table, W, b):
    users3 = users.astype(jnp.int32).reshape(NW, N_CHUNKS, IDX_CHUNK)
    items3 = items.astype(jnp.int32).reshape(NW, N_CHUNKS, IDX_CHUNK)
    w_flat = W.reshape(EMB).astype(jnp.float32)
    b_vec = jnp.broadcast_to(b.astype(jnp.float32), (LANES,))
    out = _gmf(users3, items3, user_table, item_table, w_flat, b_vec)
    return out.reshape(BATCH, 1)
